# baseline (speedup 1.0000x reference)
import functools

import jax
import jax.numpy as jnp
from jax import lax
from jax.experimental import pallas as pl
from jax.experimental.pallas import tpu as pltpu
from jax.experimental.pallas import tpu_sc as plsc

NC, NS, L = 2, 16, 16
NW = NC * NS
B = 16384
D = 64
BPW = B // NW
CH = 128
NCH = BPW // CH

_mesh = plsc.VectorSubcoreMesh(
    core_axis_name="c", subcore_axis_name="s", num_cores=NC, num_subcores=NS
)


@functools.partial(
    pl.kernel,
    out_type=jax.ShapeDtypeStruct((B,), jnp.float32),
    mesh=_mesh,
    compiler_params=pltpu.CompilerParams(
        needs_layout_passes=False, use_tc_tiling_on_sc=True
    ),
    scratch_types=[
        pltpu.VMEM((NCH, CH), jnp.int32),
        pltpu.VMEM((NCH, CH), jnp.int32),
        pltpu.VMEM((D, CH), jnp.int32),
        pltpu.VMEM((D, CH), jnp.int32),
        pltpu.VMEM((D, CH), jnp.float32),
        pltpu.VMEM((D, CH), jnp.float32),
        pltpu.VMEM((BPW,), jnp.float32),
        pltpu.VMEM((BPW,), jnp.float32),
        pltpu.VMEM((BPW,), jnp.float32),
        pltpu.SemaphoreType.DMA,
        pltpu.SemaphoreType.DMA,
    ],
)
def _mf_kernel(uidx_hbm, iidx_hbm, ut_hbm, it_hbm, ub_hbm, ib_hbm, out_hbm,
               uidx_v, iidx_v, uw_v, iw_v, utr_v, itr_v,
               ub_v, ib_v, out_v, rsem, bsem):
    wid = lax.axis_index("s") * NC + lax.axis_index("c")

    pltpu.sync_copy(uidx_hbm.at[pl.ds(wid * NCH, NCH)], uidx_v)
    pltpu.sync_copy(iidx_hbm.at[pl.ds(wid * NCH, NCH)], iidx_v)

    bias_copies = []
    for c in range(NCH):
        sl = pl.ds(c * CH, CH)
        bias_copies.append(pltpu.async_copy(ub_hbm.at[uidx_v.at[c]], ub_v.at[sl], bsem))
        bias_copies.append(pltpu.async_copy(ib_hbm.at[iidx_v.at[c]], ib_v.at[sl], bsem))
    for c in bias_copies:
        c.wait()

    for c in range(NCH):
        def build(g, carry, c=c):
            sl = pl.ds(g * L, L)
            ubase = uidx_v[c, sl] * D
            ibase = iidx_v[c, sl] * D
            for d in range(D):
                uw_v[d, sl] = ubase + d
                iw_v[d, sl] = ibase + d
            return carry

        lax.fori_loop(0, CH // L, build, 0)

        copies = []
        for d in range(D):
            copies.append(pltpu.async_copy(ut_hbm.at[uw_v.at[d]], utr_v.at[d], rsem))
            copies.append(pltpu.async_copy(it_hbm.at[iw_v.at[d]], itr_v.at[d], rsem))
        for cp in copies:
            cp.wait()

        def block(t, carry, c=c):
            j0 = t * L
            g0 = c * CH + j0
            accs = [
                ub_v[pl.ds(g0, L)] + ib_v[pl.ds(g0, L)],
                jnp.zeros((L,), jnp.float32),
                jnp.zeros((L,), jnp.float32),
                jnp.zeros((L,), jnp.float32),
            ]
            for d in range(D):
                accs[d % 4] = accs[d % 4] + utr_v[d, pl.ds(j0, L)] * itr_v[d, pl.ds(j0, L)]
            out_v[pl.ds(g0, L)] = (accs[0] + accs[1]) + (accs[2] + accs[3])
            return carry

        lax.fori_loop(0, CH // L, block, 0)

    pltpu.sync_copy(out_v, out_hbm.at[pl.ds(wid * BPW, BPW)])


def kernel(inputs, user_table, item_table, user_bias, item_bias):
    uidx = inputs[:, 0].reshape(NW * NCH, CH)
    iidx = inputs[:, 1].reshape(NW * NCH, CH)
    out = _mf_kernel(
        uidx, iidx,
        user_table.reshape(-1), item_table.reshape(-1),
        user_bias.reshape(-1), item_bias.reshape(-1),
    )
    return out.reshape(B, 1)


# speedup vs baseline: 1.4257x; 1.4257x over previous
import functools

import jax
import jax.numpy as jnp
from jax import lax
from jax.experimental import pallas as pl
from jax.experimental.pallas import tpu as pltpu
from jax.experimental.pallas import tpu_sc as plsc

NC, NS, L = 2, 16, 16
NW = NC * NS
B = 16384
D = 64
BPW = B // NW
CH = 128
NCH = BPW // CH

_mesh = plsc.VectorSubcoreMesh(
    core_axis_name="c", subcore_axis_name="s", num_cores=NC, num_subcores=NS
)


@functools.partial(
    pl.kernel,
    out_type=jax.ShapeDtypeStruct((B,), jnp.float32),
    mesh=_mesh,
    compiler_params=pltpu.CompilerParams(
        needs_layout_passes=False, use_tc_tiling_on_sc=True
    ),
    scratch_types=[
        pltpu.VMEM((NCH, CH), jnp.int32),
        pltpu.VMEM((NCH, CH), jnp.int32),
        pltpu.VMEM((CH, D), jnp.float32),
        pltpu.VMEM((CH, D), jnp.float32),
        pltpu.VMEM((BPW,), jnp.float32),
        pltpu.VMEM((BPW,), jnp.float32),
        pltpu.VMEM((BPW,), jnp.float32),
        pltpu.SemaphoreType.DMA,
        pltpu.SemaphoreType.DMA,
    ],
)
def _mf_kernel(uidx_hbm, iidx_hbm, ut_hbm, it_hbm, ub_hbm, ib_hbm, out_hbm,
               uidx_v, iidx_v, urows_v, irows_v,
               ub_v, ib_v, out_v, rsem, bsem):
    wid = lax.axis_index("s") * NC + lax.axis_index("c")

    pltpu.sync_copy(uidx_hbm.at[pl.ds(wid * NCH, NCH)], uidx_v)
    pltpu.sync_copy(iidx_hbm.at[pl.ds(wid * NCH, NCH)], iidx_v)

    bias_copies = []
    for c in range(NCH):
        sl = pl.ds(c * CH, CH)
        bias_copies.append(pltpu.async_copy(ub_hbm.at[uidx_v.at[c]], ub_v.at[sl], bsem))
        bias_copies.append(pltpu.async_copy(ib_hbm.at[iidx_v.at[c]], ib_v.at[sl], bsem))
    for c in bias_copies:
        c.wait()

    lanes = lax.iota(jnp.int32, L)

    def fire_and_wait(c):
        def group(g, carry):
            uvec = uidx_v[c, pl.ds(g * L, L)]
            ivec = iidx_v[c, pl.ds(g * L, L)]
            copies = []
            for k in range(L):
                u = uvec[k]
                v = ivec[k]
                dst = pl.ds(g * L + k, 1)
                copies.append(pltpu.async_copy(ut_hbm.at[pl.ds(u, 1)], urows_v.at[dst], rsem))
                copies.append(pltpu.async_copy(it_hbm.at[pl.ds(v, 1)], irows_v.at[dst], rsem))
            for cp in copies:
                cp.wait()
            return carry

        lax.fori_loop(0, CH // L, group, 0)

    def compute(c):
        def block(t, carry, c=c):
            b0 = t * L
            rows = b0 + lanes
            g0 = c * CH + b0
            accs = [
                ub_v[pl.ds(g0, L)] + ib_v[pl.ds(g0, L)],
                jnp.zeros((L,), jnp.float32),
                jnp.zeros((L,), jnp.float32),
                jnp.zeros((L,), jnp.float32),
            ]
            for d in range(D):
                col = jnp.full((L,), d, jnp.int32)
                uu = plsc.load_gather(urows_v, [rows, col])
                vv = plsc.load_gather(irows_v, [rows, col])
                accs[d % 4] = accs[d % 4] + uu * vv
            out_v[pl.ds(g0, L)] = (accs[0] + accs[1]) + (accs[2] + accs[3])
            return carry

        lax.fori_loop(0, CH // L, block, 0)

    for c in range(NCH):
        fire_and_wait(c)
        compute(c)

    pltpu.sync_copy(out_v, out_hbm.at[pl.ds(wid * BPW, BPW)])


def kernel(inputs, user_table, item_table, user_bias, item_bias):
    uidx = inputs[:, 0].reshape(NW * NCH, CH)
    iidx = inputs[:, 1].reshape(NW * NCH, CH)
    out = _mf_kernel(
        uidx, iidx, user_table, item_table,
        user_bias.reshape(-1), item_bias.reshape(-1),
    )
    return out.reshape(B, 1)
